# baseline (speedup 1.0000x reference)
import functools

import jax
import jax.numpy as jnp
from jax import lax
from jax.experimental import pallas as pl
from jax.experimental.pallas import tpu as pltpu
from jax.experimental.pallas import tpu_sc as plsc

B, D, H = 4096, 1024, 2048
BM = 256
BC = 256
NBLK = B // BC + 1
BP = NBLK * BC
OP = 16
OW = 128
NC, NS = 2, 16
NW = NC * NS
CHUNK = B // NW
SUB = 64


def _route_body(x_ref, p_ref, t_ref):
    xb = x_ref[...]
    p = p_ref[...]
    diff0 = xb - p[0:1, :]
    diff1 = xb - p[1:2, :]
    d0 = jnp.sqrt(jnp.sum(diff0 * diff0, axis=1, keepdims=True))
    d1 = jnp.sqrt(jnp.sum(diff1 * diff1, axis=1, keepdims=True))
    t_ref[...] = (d1 < d0).astype(jnp.int32)


def _sc_scatter_body(x_hbm, t_hbm, xs_hbm, dst_hbm, be_hbm,
                     tball, mytbuf, dstbuf, idx_v, rowbuf, bebuf, sem):
    wid = lax.axis_index("s") * NC + lax.axis_index("c")
    my_first_vreg = wid * (CHUNK // 16)

    pltpu.sync_copy(t_hbm, tball)
    pltpu.sync_copy(t_hbm.at[pl.ds(wid * CHUNK, CHUNK)], mytbuf)

    zero = jnp.zeros((16,), jnp.int32)
    acc_before = zero
    acc_total = zero
    for v in range(B // 16):
        t16 = tball[pl.ds(v * 16, 16)]
        pred = my_first_vreg > v
        acc_before = acc_before + jnp.where(pred, t16, zero)
        acc_total = acc_total + t16
    ones_before = jnp.sum(acc_before)
    n1 = jnp.sum(acc_total)
    n0 = B - n1
    nb0 = (n0 + BC - 1) // BC
    off1 = nb0 * BC

    carry = ones_before
    for v in range(CHUNK // 16):
        t16 = mytbuf[pl.ds(v * 16, 16)]
        incl = jnp.cumsum(t16)
        c1x = carry + incl - t16
        ilin = wid * CHUNK + v * 16 + lax.iota(jnp.int32, 16)
        c0x = ilin - c1x
        dstbuf[pl.ds(v * 16, 16)] = jnp.where(t16 > 0, off1 + c1x, c0x)
        carry = carry + jnp.sum(t16)

    pltpu.sync_copy(dstbuf, dst_hbm.at[pl.ds(wid * CHUNK, CHUNK)])

    for s_i in range(CHUNK // SUB):
        for u in range(SUB // 16):
            idx_v[pl.ds(u * 16, 16)] = dstbuf[pl.ds(s_i * SUB + u * 16, 16)]
        pltpu.sync_copy(x_hbm.at[pl.ds(wid * CHUNK + s_i * SUB, SUB)], rowbuf)
        pltpu.async_copy(rowbuf, xs_hbm.at[idx_v], sem).wait()

    @pl.when(wid == 0)
    def _():
        g = lax.iota(jnp.int32, 16)
        bebuf[pl.ds(0, 16)] = (g >= nb0).astype(jnp.int32)
        bebuf[pl.ds(16, 16)] = ((g + 16) >= nb0).astype(jnp.int32)
        pltpu.sync_copy(bebuf, be_hbm)


def _moe_body(be_ref, xs_ref, w1_ref, b1_ref, w2_ref, b2_ref, o_ref):
    xb = xs_ref[...]
    dn = (((1,), (1,)), ((), ()))
    h = jax.nn.relu(
        jax.lax.dot_general(xb, w1_ref[0], dn, preferred_element_type=jnp.float32)
        + b1_ref[0])
    o16 = (jax.lax.dot_general(h, w2_ref[0], dn, preferred_element_type=jnp.float32)
           + b2_ref[0])
    o_ref[...] = jnp.concatenate(
        [o16, jnp.zeros((BC, OW - OP), jnp.float32)], axis=1)


def _sc_gather_body(os_hbm, dst_hbm, out_hbm, dstbuf, rows, sem):
    wid = lax.axis_index("s") * NC + lax.axis_index("c")
    pltpu.sync_copy(dst_hbm.at[pl.ds(wid * CHUNK, CHUNK)], dstbuf)
    pltpu.async_copy(os_hbm.at[dstbuf], rows, sem).wait()
    pltpu.sync_copy(rows, out_hbm.at[pl.ds(wid * CHUNK, CHUNK)])


_sc_mesh = plsc.VectorSubcoreMesh(core_axis_name="c", subcore_axis_name="s")

_sc_params = pltpu.CompilerParams(needs_layout_passes=False)

_sc_scatter = functools.partial(
    pl.kernel, _sc_scatter_body, mesh=_sc_mesh,
    compiler_params=_sc_params,
    out_type=[
        jax.ShapeDtypeStruct((BP, D), jnp.float32),
        jax.ShapeDtypeStruct((B,), jnp.int32),
        jax.ShapeDtypeStruct((NW,), jnp.int32),
    ],
    scratch_types=[
        pltpu.VMEM((B,), jnp.int32),
        pltpu.VMEM((CHUNK,), jnp.int32),
        pltpu.VMEM((CHUNK,), jnp.int32),
        pltpu.VMEM((SUB,), jnp.int32),
        pltpu.VMEM((SUB, D), jnp.float32),
        pltpu.VMEM((NW,), jnp.int32),
        pltpu.SemaphoreType.DMA,
    ],
)

_sc_gather = functools.partial(
    pl.kernel, _sc_gather_body, mesh=_sc_mesh,
    compiler_params=_sc_params,
    out_type=[jax.ShapeDtypeStruct((B, OW), jnp.float32)],
    scratch_types=[
        pltpu.VMEM((CHUNK,), jnp.int32),
        pltpu.VMEM((CHUNK, OW), jnp.float32),
        pltpu.SemaphoreType.DMA,
    ],
)


def kernel(x, w1, b1, w2, b2, prototypes):
    w2p = jnp.zeros((2, OP, H), jnp.float32).at[:, :2, :].set(w2)
    b2p = jnp.zeros((2, 1, OP), jnp.float32).at[:, 0, :2].set(b2)
    b1r = b1.reshape(2, 1, H)

    t2d = pl.pallas_call(
        _route_body,
        grid=(B // BM,),
        in_specs=[
            pl.BlockSpec((BM, D), lambda i: (i, 0)),
            pl.BlockSpec((2, D), lambda i: (0, 0)),
        ],
        out_specs=pl.BlockSpec((BM, 1), lambda i: (i, 0)),
        out_shape=jax.ShapeDtypeStruct((B, 1), jnp.int32),
    )(x, prototypes)
    t_flat = t2d.reshape(B)

    xs, dst, be = _sc_scatter()(x, t_flat)

    os = pl.pallas_call(
        _moe_body,
        grid_spec=pltpu.PrefetchScalarGridSpec(
            num_scalar_prefetch=1,
            grid=(NBLK,),
            in_specs=[
                pl.BlockSpec((BC, D), lambda g, be_r: (g, 0)),
                pl.BlockSpec((1, H, D), lambda g, be_r: (be_r[g], 0, 0)),
                pl.BlockSpec((1, 1, H), lambda g, be_r: (be_r[g], 0, 0)),
                pl.BlockSpec((1, OP, H), lambda g, be_r: (be_r[g], 0, 0)),
                pl.BlockSpec((1, 1, OP), lambda g, be_r: (be_r[g], 0, 0)),
            ],
            out_specs=pl.BlockSpec((BC, OW), lambda g, be_r: (g, 0)),
        ),
        out_shape=jax.ShapeDtypeStruct((BP, OW), jnp.float32),
    )(be, xs, w1, b1r, w2p, b2p)

    (out_pad,) = _sc_gather()(os, dst)
    return out_pad[:, :2]


# speedup vs baseline: 1.0877x; 1.0877x over previous
import functools

import jax
import jax.numpy as jnp
from jax import lax
from jax.experimental import pallas as pl
from jax.experimental.pallas import tpu as pltpu
from jax.experimental.pallas import tpu_sc as plsc

B, D, H = 4096, 1024, 2048
BM = 512
BC = 512
NBLK = B // BC + 1
BP = NBLK * BC
OW = 128
NC, NS = 2, 16
NW = NC * NS
CHUNK = B // NW
SUB = 32


def _route_body(x_ref, p_ref, t_ref):
    xb = x_ref[...]
    p = p_ref[...]
    diff0 = xb - p[0:1, :]
    diff1 = xb - p[1:2, :]
    d0 = jnp.sqrt(jnp.sum(diff0 * diff0, axis=1, keepdims=True))
    d1 = jnp.sqrt(jnp.sum(diff1 * diff1, axis=1, keepdims=True))
    t_ref[...] = (d1 < d0).astype(jnp.int32)


def _sc_scatter_body(x_hbm, t_hbm, xs_hbm, dst_hbm, be_hbm,
                     tball, mytbuf, dstbuf, bufa, bufb, bufc,
                     idxa, idxb, idxc, bebuf,
                     sema, semb, semc):
    wid = lax.axis_index("s") * NC + lax.axis_index("c")
    my_first_vreg = wid * (CHUNK // 16)
    base = wid * CHUNK

    in_a = pltpu.async_copy(x_hbm.at[pl.ds(base + 0 * SUB, SUB)], bufa, sema)
    in_b = pltpu.async_copy(x_hbm.at[pl.ds(base + 1 * SUB, SUB)], bufb, semb)
    in_c = pltpu.async_copy(x_hbm.at[pl.ds(base + 2 * SUB, SUB)], bufc, semc)

    pltpu.sync_copy(t_hbm, tball)
    pltpu.sync_copy(t_hbm.at[pl.ds(base, CHUNK)], mytbuf)

    zero = jnp.zeros((16,), jnp.int32)
    acc_before = zero
    acc_total = zero
    for v in range(B // 16):
        t16 = tball[pl.ds(v * 16, 16)]
        pred = my_first_vreg > v
        acc_before = acc_before + jnp.where(pred, t16, zero)
        acc_total = acc_total + t16
    ones_before = jnp.sum(acc_before)
    n1 = jnp.sum(acc_total)
    n0 = B - n1
    nb0 = (n0 + BC - 1) // BC
    off1 = nb0 * BC

    carry = ones_before
    for v in range(CHUNK // 16):
        t16 = mytbuf[pl.ds(v * 16, 16)]
        incl = jnp.cumsum(t16)
        c1x = carry + incl - t16
        ilin = base + v * 16 + lax.iota(jnp.int32, 16)
        c0x = ilin - c1x
        dstbuf[pl.ds(v * 16, 16)] = jnp.where(t16 > 0, off1 + c1x, c0x)
        carry = carry + jnp.sum(t16)

    pltpu.sync_copy(dstbuf, dst_hbm.at[pl.ds(base, CHUNK)])

    def fill_idx(idx_v, s_i):
        for u in range(SUB // 16):
            idx_v[pl.ds(u * 16, 16)] = dstbuf[pl.ds(s_i * SUB + u * 16, 16)]

    fill_idx(idxa, 0)
    fill_idx(idxb, 1)
    fill_idx(idxc, 2)
    in_a.wait()
    out_a = pltpu.async_copy(bufa, xs_hbm.at[idxa], sema)
    in_b.wait()
    out_b = pltpu.async_copy(bufb, xs_hbm.at[idxb], semb)
    in_c.wait()
    out_c = pltpu.async_copy(bufc, xs_hbm.at[idxc], semc)
    out_a.wait()
    in_a2 = pltpu.async_copy(x_hbm.at[pl.ds(base + 3 * SUB, SUB)], bufa, sema)
    fill_idx(idxa, 3)
    in_a2.wait()
    out_a2 = pltpu.async_copy(bufa, xs_hbm.at[idxa], sema)
    out_b.wait()
    out_c.wait()
    out_a2.wait()

    @pl.when(wid == 0)
    def _():
        g = lax.iota(jnp.int32, 16)
        bebuf[pl.ds(0, 16)] = (g >= nb0).astype(jnp.int32)
        bebuf[pl.ds(16, 16)] = ((g + 16) >= nb0).astype(jnp.int32)
        pltpu.sync_copy(bebuf, be_hbm)


def _moe_body(be_ref, xs_ref, w1_ref, b1_ref, w2_ref, b2_ref, o_ref):
    xb = xs_ref[...]
    dn = (((1,), (1,)), ((), ()))
    h = jax.nn.relu(
        jax.lax.dot_general(xb, w1_ref[0], dn, preferred_element_type=jnp.float32)
        + b1_ref[0])
    o2 = (jax.lax.dot_general(h, w2_ref[0], dn, preferred_element_type=jnp.float32)
          + b2_ref[0])
    o_ref[...] = jnp.concatenate(
        [o2, jnp.zeros((BC, OW - 2), jnp.float32)], axis=1)


def _sc_gather_body(os_hbm, dst_hbm, out_hbm, dstbuf, rows, sem):
    wid = lax.axis_index("s") * NC + lax.axis_index("c")
    pltpu.sync_copy(dst_hbm.at[pl.ds(wid * CHUNK, CHUNK)], dstbuf)
    pltpu.async_copy(os_hbm.at[dstbuf], rows, sem).wait()
    pltpu.sync_copy(rows, out_hbm.at[pl.ds(wid * CHUNK, CHUNK)])


_sc_mesh = plsc.VectorSubcoreMesh(core_axis_name="c", subcore_axis_name="s")

_sc_params = pltpu.CompilerParams(needs_layout_passes=False)

_sc_scatter = functools.partial(
    pl.kernel, _sc_scatter_body, mesh=_sc_mesh,
    compiler_params=_sc_params,
    out_type=[
        jax.ShapeDtypeStruct((BP, D), jnp.float32),
        jax.ShapeDtypeStruct((B,), jnp.int32),
        jax.ShapeDtypeStruct((NW,), jnp.int32),
    ],
    scratch_types=[
        pltpu.VMEM((B,), jnp.int32),
        pltpu.VMEM((CHUNK,), jnp.int32),
        pltpu.VMEM((CHUNK,), jnp.int32),
        pltpu.VMEM((SUB, D), jnp.float32),
        pltpu.VMEM((SUB, D), jnp.float32),
        pltpu.VMEM((SUB, D), jnp.float32),
        pltpu.VMEM((SUB,), jnp.int32),
        pltpu.VMEM((SUB,), jnp.int32),
        pltpu.VMEM((SUB,), jnp.int32),
        pltpu.VMEM((NW,), jnp.int32),
        pltpu.SemaphoreType.DMA,
        pltpu.SemaphoreType.DMA,
        pltpu.SemaphoreType.DMA,
    ],
)

_sc_gather = functools.partial(
    pl.kernel, _sc_gather_body, mesh=_sc_mesh,
    compiler_params=_sc_params,
    out_type=[jax.ShapeDtypeStruct((B, OW), jnp.float32)],
    scratch_types=[
        pltpu.VMEM((CHUNK,), jnp.int32),
        pltpu.VMEM((CHUNK, OW), jnp.float32),
        pltpu.SemaphoreType.DMA,
    ],
)


def kernel(x, w1, b1, w2, b2, prototypes):
    b1r = b1.reshape(2, 1, H)
    b2r = b2.reshape(2, 1, 2)

    t2d = pl.pallas_call(
        _route_body,
        grid=(B // BM,),
        in_specs=[
            pl.BlockSpec((BM, D), lambda i: (i, 0)),
            pl.BlockSpec((2, D), lambda i: (0, 0)),
        ],
        out_specs=pl.BlockSpec((BM, 1), lambda i: (i, 0)),
        out_shape=jax.ShapeDtypeStruct((B, 1), jnp.int32),
    )(x, prototypes)
    t_flat = t2d.reshape(B)

    xs, dst, be = _sc_scatter()(x, t_flat)

    os = pl.pallas_call(
        _moe_body,
        grid_spec=pltpu.PrefetchScalarGridSpec(
            num_scalar_prefetch=1,
            grid=(NBLK,),
            in_specs=[
                pl.BlockSpec((BC, D), lambda g, be_r: (g, 0)),
                pl.BlockSpec((1, H, D), lambda g, be_r: (be_r[g], 0, 0)),
                pl.BlockSpec((1, 1, H), lambda g, be_r: (be_r[g], 0, 0)),
                pl.BlockSpec((1, 2, H), lambda g, be_r: (be_r[g], 0, 0)),
                pl.BlockSpec((1, 1, 2), lambda g, be_r: (be_r[g], 0, 0)),
            ],
            out_specs=pl.BlockSpec((BC, OW), lambda g, be_r: (g, 0)),
        ),
        out_shape=jax.ShapeDtypeStruct((BP, OW), jnp.float32),
    )(be, xs, w1, b1r, w2, b2r)

    (out_pad,) = _sc_gather()(os, dst)
    return out_pad[:, :2]


# speedup vs baseline: 1.5842x; 1.4565x over previous
import jax
import jax.numpy as jnp
from jax.experimental import pallas as pl

B, D, H = 4096, 1024, 2048
BM = 512


def _dense_body(x_ref, w1_ref, b1_ref, w2_ref, b2_ref, p_ref, o_ref):
    xb = x_ref[...]
    p = p_ref[...]
    diff0 = xb - p[0:1, :]
    diff1 = xb - p[1:2, :]
    d0 = jnp.sqrt(jnp.sum(diff0 * diff0, axis=1, keepdims=True))
    d1 = jnp.sqrt(jnp.sum(diff1 * diff1, axis=1, keepdims=True))
    pick1 = d1 < d0

    dn = (((1,), (1,)), ((), ()))
    h0 = jax.nn.relu(
        jax.lax.dot_general(xb, w1_ref[0], dn, preferred_element_type=jnp.float32)
        + b1_ref[0])
    o0 = (jax.lax.dot_general(h0, w2_ref[0], dn, preferred_element_type=jnp.float32)
          + b2_ref[0])
    h1 = jax.nn.relu(
        jax.lax.dot_general(xb, w1_ref[1], dn, preferred_element_type=jnp.float32)
        + b1_ref[1])
    o1 = (jax.lax.dot_general(h1, w2_ref[1], dn, preferred_element_type=jnp.float32)
          + b2_ref[1])
    o_ref[...] = jnp.where(pick1, o1, o0)


def kernel(x, w1, b1, w2, b2, prototypes):
    b1r = b1.reshape(2, 1, H)
    b2r = b2.reshape(2, 1, 2)
    out = pl.pallas_call(
        _dense_body,
        grid=(B // BM,),
        in_specs=[
            pl.BlockSpec((BM, D), lambda i: (i, 0)),
            pl.BlockSpec((2, H, D), lambda i: (0, 0, 0)),
            pl.BlockSpec((2, 1, H), lambda i: (0, 0, 0)),
            pl.BlockSpec((2, 2, H), lambda i: (0, 0, 0)),
            pl.BlockSpec((2, 1, 2), lambda i: (0, 0, 0)),
            pl.BlockSpec((2, D), lambda i: (0, 0)),
        ],
        out_specs=pl.BlockSpec((BM, 2), lambda i: (i, 0)),
        out_shape=jax.ShapeDtypeStruct((B, 2), jnp.float32),
    )(x, w1, b1r, w2, b2r, prototypes)
    return out


# speedup vs baseline: 1.5913x; 1.0045x over previous
import jax
import jax.numpy as jnp
from jax.experimental import pallas as pl

B, D, H = 4096, 1024, 2048
BM = 1024


def _dense_body(x_ref, w1_ref, b1_ref, w2_ref, b2_ref, p_ref, o_ref):
    xb = x_ref[...]
    p = p_ref[...]
    diff0 = xb - p[0:1, :]
    diff1 = xb - p[1:2, :]
    d0 = jnp.sqrt(jnp.sum(diff0 * diff0, axis=1, keepdims=True))
    d1 = jnp.sqrt(jnp.sum(diff1 * diff1, axis=1, keepdims=True))
    pick1 = d1 < d0

    dn = (((1,), (1,)), ((), ()))
    h0 = jax.nn.relu(
        jax.lax.dot_general(xb, w1_ref[0], dn, preferred_element_type=jnp.float32)
        + b1_ref[0])
    o0 = (jax.lax.dot_general(h0, w2_ref[0], dn, preferred_element_type=jnp.float32)
          + b2_ref[0])
    h1 = jax.nn.relu(
        jax.lax.dot_general(xb, w1_ref[1], dn, preferred_element_type=jnp.float32)
        + b1_ref[1])
    o1 = (jax.lax.dot_general(h1, w2_ref[1], dn, preferred_element_type=jnp.float32)
          + b2_ref[1])
    o_ref[...] = jnp.where(pick1, o1, o0)


def kernel(x, w1, b1, w2, b2, prototypes):
    b1r = b1.reshape(2, 1, H)
    b2r = b2.reshape(2, 1, 2)
    out = pl.pallas_call(
        _dense_body,
        grid=(B // BM,),
        in_specs=[
            pl.BlockSpec((BM, D), lambda i: (i, 0)),
            pl.BlockSpec((2, H, D), lambda i: (0, 0, 0)),
            pl.BlockSpec((2, 1, H), lambda i: (0, 0, 0)),
            pl.BlockSpec((2, 2, H), lambda i: (0, 0, 0)),
            pl.BlockSpec((2, 1, 2), lambda i: (0, 0, 0)),
            pl.BlockSpec((2, D), lambda i: (0, 0)),
        ],
        out_specs=pl.BlockSpec((BM, 2), lambda i: (i, 0)),
        out_shape=jax.ShapeDtypeStruct((B, 2), jnp.float32),
    )(x, w1, b1r, w2, b2r, prototypes)
    return out
